# baseline (speedup 1.0000x reference)
import functools

import jax
import jax.numpy as jnp
from jax import lax
from jax.experimental import pallas as pl
from jax.experimental.pallas import tpu as pltpu
from jax.experimental.pallas import tpu_sc as plsc

NUM_EMB = 1000000
D = 64
DP = 128
B = 4096
H = 50

NC = 2
NS = 16
NW = NC * NS
L = 16

BPW = B // NW
BPC = 2
CROWS = BPC * H
NCHUNK = BPW // BPC
NBUF = 2
ND = D // L

SCBLK = 156
SCR = NW * SCBLK * 128
X_TC = NUM_EMB - SCR

_mesh = plsc.VectorSubcoreMesh(core_axis_name="c", subcore_axis_name="s")


_BLK = 4096
_NGRID = (X_TC + 8 + _BLK - 1) // _BLK


def _relayout_body(tin_ref, tout_ref):
    i = pl.program_id(0)
    grow = i * _BLK + lax.broadcasted_iota(jnp.int32, (_BLK, 1), 0)
    tout_ref[:, 0:D] = jnp.where(grow < X_TC, tin_ref[...].T, 0.0)


_relayout = pl.pallas_call(
    _relayout_body,
    grid=(_NGRID,),
    in_specs=[pl.BlockSpec((D, _BLK), lambda i: (0, i + SCR // _BLK))],
    out_specs=pl.BlockSpec((_BLK, DP), lambda i: (i, 0)),
    out_shape=jax.ShapeDtypeStruct((X_TC + 8, DP), jnp.float32),
)


@functools.partial(
    pl.kernel,
    out_type=jax.ShapeDtypeStruct((SCR + 8, DP), jnp.float32),
    mesh=_mesh,
    scratch_types=[
        pltpu.VMEM((D, DP), jnp.float32),
        pltpu.VMEM((D, DP), jnp.float32),
        pltpu.VMEM((DP, DP), jnp.float32),
        pltpu.VMEM((DP, DP), jnp.float32),
        pltpu.SemaphoreType.DMA,
        pltpu.SemaphoreType.DMA,
        pltpu.SemaphoreType.DMA,
        pltpu.SemaphoreType.DMA,
    ],
    compiler_params=pltpu.CompilerParams(
        use_tc_tiling_on_sc=True, needs_layout_passes=False),
)
def _sc_relayout(tt, out, ib0, ib1, ob0, ob1, is0, is1, os0, os1):
    w = lax.axis_index("c") * NS + lax.axis_index("s")
    b0 = w * SCBLK
    ibs, obs, isems, osems = (ib0, ib1), (ob0, ob1), (is0, is1), (os0, os1)
    dq = [lax.iota(jnp.int32, 16) + 16 * q for q in range(ND)]

    def in_copy(k, s):
        return pltpu.make_async_copy(
            tt.at[:, pl.ds((b0 + k) * DP, DP)], ibs[s], isems[s])

    def out_copy(k, s):
        return pltpu.make_async_copy(
            obs[s], out.at[pl.ds((b0 + k) * DP, DP), :], osems[s])

    for s in (0, 1):
        in_copy(s, s).start()

    def step(i, carry):
        for s in (0, 1):
            k = i * 2 + s
            in_copy(k, s).wait()

            @pl.when(k >= 2)
            def _drain(k=k, s=s):
                out_copy(k - 2, s).wait()

            for r in range(DP):
                rv = jnp.full((16,), r, jnp.int32)
                for q in range(ND):
                    obs[s][r, pl.ds(16 * q, 16)] = plsc.load_gather(
                        ibs[s], [dq[q], rv])
            out_copy(k, s).start()

            @pl.when(k + 2 < SCBLK)
            def _next(k=k, s=s):
                in_copy(k + 2, s).start()
        return carry

    lax.fori_loop(0, SCBLK // 2, step, 0)
    for s in (0, 1):
        out_copy(SCBLK - 2 + s, s).wait()

    @pl.when(w == NW - 1)
    def _zeros():
        z = jnp.zeros((16,), jnp.float32)
        for r in range(8):
            for q in range(DP // 16):
                ob0[r, pl.ds(16 * q, 16)] = z
        pltpu.sync_copy(ob0.at[pl.ds(0, 8), :], out.at[pl.ds(SCR, 8), :])


@functools.partial(
    pl.kernel,
    out_type=jax.ShapeDtypeStruct((B, D), jnp.float32),
    mesh=_mesh,
    scratch_types=[
        pltpu.VMEM((NCHUNK, CROWS), jnp.int32),
        pltpu.VMEM((NCHUNK, CROWS), jnp.int32),
        pltpu.VMEM((NBUF, CROWS, DP), jnp.float32),
        pltpu.VMEM((NBUF, CROWS, DP), jnp.float32),
        pltpu.VMEM((BPW, D), jnp.float32),
        pltpu.SemaphoreType.DMA,
        pltpu.SemaphoreType.DMA,
        pltpu.SemaphoreType.DMA,
        pltpu.SemaphoreType.DMA,
    ],
    compiler_params=pltpu.CompilerParams(use_tc_tiling_on_sc=False),
)
def _embbag(i1_2d, i2_2d, tp1, tp2, out, idx1_v, idx2_v, ring1, ring2,
            out_v, s10, s11, s20, s21):
    sems1, sems2 = (s10, s11), (s20, s21)
    w = lax.axis_index("c") * NS + lax.axis_index("s")
    inv = jnp.float32(1.0 / H)

    pltpu.sync_copy(i1_2d.at[pl.ds(w * NCHUNK, NCHUNK), :], idx1_v)
    pltpu.sync_copy(i2_2d.at[pl.ds(w * NCHUNK, NCHUNK), :], idx2_v)

    def g1(g, b):
        return pltpu.make_async_copy(
            tp1.at[idx1_v.at[g]], ring1.at[b], sems1[b])

    def g2(g, b):
        return pltpu.make_async_copy(
            tp2.at[idx2_v.at[g]], ring2.at[b], sems2[b])

    for b in range(NBUF):
        g1(b, b).start()
        g2(b, b).start()

    def step(i, carry):
        for b in range(NBUF):
            g = i * NBUF + b
            g1(g, b).wait()
            g2(g, b).wait()
            for bb in range(BPC):
                r0 = bb * H
                accs = [[ring1[b, r0 + k, pl.ds(L * d, L)] for k in range(2)]
                        for d in range(ND)]
                for j in range(2, H, 2):
                    for d in range(ND):
                        for k in range(2):
                            accs[d][k] = accs[d][k] + ring1[
                                b, r0 + j + k, pl.ds(L * d, L)]
                for j in range(0, H, 2):
                    for d in range(ND):
                        for k in range(2):
                            accs[d][k] = accs[d][k] + ring2[
                                b, r0 + j + k, pl.ds(L * d, L)]
                row = g * BPC + bb
                for d in range(ND):
                    out_v[row, pl.ds(L * d, L)] = (accs[d][0] + accs[d][1]) * inv
            ng = g + NBUF

            @pl.when(ng < NCHUNK)
            def _start(ng=ng, b=b):
                g1(ng, b).start()
                g2(ng, b).start()
        return carry

    lax.fori_loop(0, NCHUNK // NBUF, step, 0)
    pltpu.sync_copy(out_v, out.at[pl.ds(w * BPW, BPW), :])


def kernel(x, table):
    xi = x.astype(jnp.int32)
    i1 = jnp.where(xi >= SCR, xi - SCR, X_TC).reshape(B * H // CROWS, CROWS)
    i2 = jnp.where(xi < SCR, xi, SCR).reshape(B * H // CROWS, CROWS)
    tt = table.T
    tp1 = _relayout(tt)
    tp2 = _sc_relayout(tt)
    return _embbag(i1, i2, tp1, tp2)


# speedup vs baseline: 17.8929x; 17.8929x over previous
import functools

import jax
import jax.numpy as jnp
from jax import lax
from jax.experimental import pallas as pl
from jax.experimental.pallas import tpu as pltpu
from jax.experimental.pallas import tpu_sc as plsc

NUM_EMB = 1000000
D = 64
DP = 128
B = 4096
H = 50

NC = 2
NS = 16
NW = NC * NS
L = 16

BPW = B // NW
BPC = 2
CROWS = BPC * H
NCHUNK = BPW // BPC
NBUF = 2
ND = D // L

_mesh = plsc.VectorSubcoreMesh(core_axis_name="c", subcore_axis_name="s")


_BLK = 8192
_NGRID = (NUM_EMB + _BLK - 1) // _BLK


def _relayout_body(tin_ref, tout_ref):
    tout_ref[:, 0:D] = tin_ref[...].T


_relayout = pl.pallas_call(
    _relayout_body,
    grid=(_NGRID,),
    in_specs=[pl.BlockSpec((D, _BLK), lambda i: (0, i))],
    out_specs=pl.BlockSpec((_BLK, DP), lambda i: (i, 0)),
    out_shape=jax.ShapeDtypeStruct((NUM_EMB, DP), jnp.float32),
)


@functools.partial(
    pl.kernel,
    out_type=jax.ShapeDtypeStruct((B, D), jnp.float32),
    mesh=_mesh,
    scratch_types=[
        pltpu.VMEM((NCHUNK, CROWS), jnp.int32),
        pltpu.VMEM((NBUF, CROWS, DP), jnp.float32),
        pltpu.VMEM((BPW, D), jnp.float32),
        pltpu.SemaphoreType.DMA,
        pltpu.SemaphoreType.DMA,
    ],
    compiler_params=pltpu.CompilerParams(use_tc_tiling_on_sc=False),
)
def _embbag(x2d, table, out, idx_v, ring_v, out_v, sem0, sem1):
    sems = (sem0, sem1)
    w = lax.axis_index("c") * NS + lax.axis_index("s")
    inv = jnp.float32(1.0 / H)

    pltpu.sync_copy(x2d.at[pl.ds(w * NCHUNK, NCHUNK), :], idx_v)

    for b in range(NBUF):
        pltpu.make_async_copy(table.at[idx_v.at[b]], ring_v.at[b], sems[b]).start()

    def step(i, carry):
        for b in range(NBUF):
            g = i * NBUF + b
            pltpu.make_async_copy(
                table.at[idx_v.at[g]], ring_v.at[b], sems[b]).wait()
            for bb in range(BPC):
                r0 = bb * H
                accs = [[ring_v[b, r0 + k, pl.ds(L * d, L)] for k in range(2)]
                        for d in range(ND)]
                for j in range(2, H, 2):
                    for d in range(ND):
                        for k in range(2):
                            accs[d][k] = accs[d][k] + ring_v[
                                b, r0 + j + k, pl.ds(L * d, L)]
                row = g * BPC + bb
                for d in range(ND):
                    out_v[row, pl.ds(L * d, L)] = (accs[d][0] + accs[d][1]) * inv
            ng = g + NBUF

            @pl.when(ng < NCHUNK)
            def _start(ng=ng, b=b):
                pltpu.make_async_copy(
                    table.at[idx_v.at[ng]], ring_v.at[b], sems[b]).start()
        return carry

    lax.fori_loop(0, NCHUNK // NBUF, step, 0)

    pltpu.sync_copy(out_v, out.at[pl.ds(w * BPW, BPW), :])


def kernel(x, table):
    x2d = x.reshape(B * H // CROWS, CROWS).astype(jnp.int32)
    tp = _relayout(table.T)
    return _embbag(x2d, tp)
